# Initial kernel scaffold; baseline (speedup 1.0000x reference)
import jax
import jax.numpy as jnp
from jax.experimental import pallas as pl


def kernel(node_input, node_attr, node_deg, edge_src, edge_dst, edge_attr, edge_length_embedded, W_lin_in, W_mask, W1, b1, W2, b2, W3, b3, W_out):
    raise NotImplementedError("write your pallas kernel here")


# speedup vs baseline: 1.3377x; 1.3377x over previous
import math

import numpy as np
import jax
import jax.numpy as jnp
from jax import lax
from jax.experimental import pallas as pl
from jax.experimental.pallas import tpu as pltpu
from jax.experimental.pallas import tpu_sc as plsc

_N = 50000
_E = 800000
_DI = 32
_DA = 32
_DE = 4
_NB = 10
_DR = 64
_DO = 64
_DM = 128

_BN = 1000
_BE = 2000
_C = 1000
_NW = 32
_EPW = _E // _NW
_EPT = _E // 16
_NH = 25000
_TR = 512
_SPR = 25600
_ZR = 1600


def _p1_body(x_ref, a_ref, dg_ref, wl_ref, wm_ref, nf_ref, nm_ref):
    x = x_ref[...]
    a = a_ref[...]
    r = lax.rsqrt(dg_ref[...])
    gl = jnp.dot(x, wl_ref[...], preferred_element_type=jnp.float32)
    gm = jnp.dot(x, wm_ref[...], preferred_element_type=jnp.float32)
    nf = a[:, 0:1] * gl[:, 0:_DI]
    nm = a[:, 0:1] * gm[:, 0:_DO]
    for j in range(1, _DA):
        aj = a[:, j:j + 1]
        nf = nf + aj * gl[:, j * _DI:(j + 1) * _DI]
        nm = nm + aj * gm[:, j * _DO:(j + 1) * _DO]
    nf_ref[...] = nf * r
    nm_ref[...] = nm


def _p2_body(nf_hbm, src_hbm, out_hbm, nf_sh, idx_ref, rows_ref, sem):
    c = lax.axis_index("c")
    s = lax.axis_index("s")
    wid = s * 2 + c
    wbase = wid * _EPW
    @pl.when(s < 15)
    def _():
        pltpu.sync_copy(nf_hbm.at[pl.ds(s * 3128, 3128)],
                        nf_sh.at[pl.ds(s * 3128, 3128)])

    @pl.when(s == 15)
    def _():
        pltpu.sync_copy(nf_hbm.at[pl.ds(15 * 3128, _N - 15 * 3128)],
                        nf_sh.at[pl.ds(15 * 3128, _N - 15 * 3128)])

    pad = wid * 16 + lax.iota(jnp.int32, 16)
    r7 = idx_ref.at[7]
    r7[pl.ds(104, 16)] = pad
    r7[pl.ds(112, 16)] = pad
    plsc.subcore_barrier()

    def chunk(i, carry):
        base = wbase + i * _C
        for j in range(7):
            pltpu.sync_copy(src_hbm.at[pl.ds(base + j * 128, 128)], idx_ref.at[j])
        pltpu.sync_copy(src_hbm.at[pl.ds(base + 896, 104)],
                        idx_ref.at[7, pl.ds(0, 104)])
        for j in range(4):
            pltpu.async_copy(nf_sh.at[idx_ref.at[j]],
                             rows_ref.at[pl.ds(j * 128, 128)], sem).wait()
        pltpu.sync_copy(rows_ref, out_hbm.at[pl.ds(base, 512)])
        for j in range(4, 8):
            pltpu.async_copy(nf_sh.at[idx_ref.at[j]],
                             rows_ref.at[pl.ds((j - 4) * 128, 128)], sem).wait()
        pltpu.sync_copy(rows_ref.at[pl.ds(0, 488)],
                        out_hbm.at[pl.ds(base + 512, 488)])
        return carry

    lax.fori_loop(0, _EPW // _C, chunk, 0)


def _p3_body(el_ref, ea_ref, sf_ref, w1_ref, b1_ref, w2_ref, b2_ref,
             w3_ref, b3_ref, rm_ref, tm_ref, ef_ref):
    h = jnp.dot(el_ref[...], w1_ref[...], preferred_element_type=jnp.float32) + b1_ref[...]
    h = h * jax.nn.sigmoid(h)
    h = jnp.dot(h, w2_ref[...], preferred_element_type=jnp.float32) + b2_ref[...]
    h = h * jax.nn.sigmoid(h)
    ew = jnp.dot(h, w3_ref[...], preferred_element_type=jnp.float32) + b3_ref[...]
    se = jnp.dot(sf_ref[...], rm_ref[...], preferred_element_type=jnp.float32)
    at = jnp.dot(ea_ref[...], tm_ref[...], preferred_element_type=jnp.float32)
    ef = se * at * ew
    ef_ref[0] = ef[:, :_DO]
    ef_ref[1] = ef[:, _DO:]


def _p4_body(ef_hbm, dst_hbm, z_hbm, out_hbm, acc_sh, ef_ref, idx_ref):
    c = lax.axis_index("c")
    s = lax.axis_index("s")
    iota = lax.iota(jnp.int32, 16)
    neg1 = jnp.full((16,), -1, jnp.int32)
    for p in range(2):
        base_p = p * _NH
        pltpu.sync_copy(z_hbm, acc_sh.at[pl.ds(s * _ZR, _ZR)])
        plsc.subcore_barrier()

        def chunk(i, carry):
            base = s * _EPT + i * _C
            for j in range(7):
                pltpu.sync_copy(dst_hbm.at[pl.ds(base + j * 128, 128)], idx_ref.at[j])
            pltpu.sync_copy(dst_hbm.at[pl.ds(base + 896, 104)],
                            idx_ref.at[7, pl.ds(0, 104)])
            r7 = idx_ref.at[7]
            r7[pl.ds(104, 16)] = neg1
            r7[pl.ds(112, 16)] = neg1
            for l in range(64):
                j, col = divmod(l, 8)
                col *= 16
                rr = idx_ref.at[j]
                v = rr[pl.ds(col, 16)]
                loc = v - base_p
                m = (loc >= 0) & (loc < _NH)
                tr = _NH + ((l * 16 + iota) & (_TR - 1))
                rr[pl.ds(col, 16)] = jnp.where(m, loc, tr)
            for q in range(4):
                if q < 3:
                    pltpu.sync_copy(ef_hbm.at[c, pl.ds(base + q * 256, 256)],
                                    ef_ref)
                else:
                    pltpu.sync_copy(ef_hbm.at[c, pl.ds(base + 768, 232)],
                                    ef_ref.at[pl.ds(0, 232)])
                pltpu.sync_copy(ef_ref.at[pl.ds(0, 128)],
                                acc_sh.at[idx_ref.at[2 * q]], add=True)
                pltpu.sync_copy(ef_ref.at[pl.ds(128, 128)],
                                acc_sh.at[idx_ref.at[2 * q + 1]], add=True)
            return carry

        lax.fori_loop(0, _EPT // _C, chunk, 0)
        plsc.subcore_barrier()

        @pl.when(s < 15)
        def _():
            pltpu.sync_copy(acc_sh.at[pl.ds(s * _ZR, _ZR)],
                            out_hbm.at[c, p, pl.ds(s * _ZR, _ZR)])

        @pl.when(s == 15)
        def _():
            pltpu.sync_copy(acc_sh.at[pl.ds(15 * _ZR, _NH - 15 * _ZR)],
                            out_hbm.at[c, p, pl.ds(15 * _ZR, _NH - 15 * _ZR)])


def _p5_body(aa_ref, ab_ref, a_ref, nm_ref, dg_ref, wa_ref, wb_ref, o_ref):
    r = lax.rsqrt(dg_ref[...])
    ga = jnp.dot(aa_ref[...] * r, wa_ref[...], preferred_element_type=jnp.float32)
    gb = jnp.dot(ab_ref[...] * r, wb_ref[...], preferred_element_type=jnp.float32)
    g = ga + gb
    a = a_ref[...]
    acc = nm_ref[...]
    for j in range(_DA):
        acc = acc + a[:, j:j + 1] * g[:, j * _DO:(j + 1) * _DO]
    o_ref[...] = acc


def kernel(node_input, node_attr, node_deg, edge_src, edge_dst, edge_attr,
           edge_length_embedded, W_lin_in, W_mask, W1, b1, W2, b2, W3, b3, W_out):
    f32 = jnp.float32
    c_s, c_x = math.sin(math.pi / 8), math.cos(math.pi / 8)

    wl = W_lin_in.reshape(_DI, _DA * _DI) * (1.0 / 32.0)
    wm = W_mask.reshape(_DI, _DA * _DO) * (c_s / 32.0)
    wo = W_out.reshape(_DM, _DA * _DO) * (c_x / 64.0)
    wa, wb = wo[:_DO], wo[_DO:]
    w3 = W3 * 0.5
    b3s = (b3 * 0.5).reshape(1, _DM)
    b1r = b1.reshape(1, _DR)
    b2r = b2.reshape(1, _DR)

    rm_np = np.zeros((_DI, _DM), np.float32)
    tm_np = np.zeros((_DE, _DM), np.float32)
    for i in range(_DI):
        rm_np[i, 4 * i:4 * i + 4] = 1.0
    for cc in range(_DE):
        tm_np[cc, cc::4] = 1.0
    rm = jnp.asarray(rm_np)
    tm = jnp.asarray(tm_np)

    grid_n = _N // _BN
    nspec = lambda w: pl.BlockSpec((_BN, w), lambda i: (i, 0))
    full = lambda *shp: pl.BlockSpec(shp, lambda i: tuple(0 for _ in shp))

    nf, nm = pl.pallas_call(
        _p1_body,
        grid=(grid_n,),
        in_specs=[nspec(_DI), nspec(_DA), nspec(1),
                  full(_DI, _DA * _DI), full(_DI, _DA * _DO)],
        out_specs=[nspec(_DI), nspec(_DO)],
        out_shape=[jax.ShapeDtypeStruct((_N, _DI), f32),
                   jax.ShapeDtypeStruct((_N, _DO), f32)],
    )(node_input, node_attr, node_deg, wl, wm)

    mesh = plsc.VectorSubcoreMesh(core_axis_name="c", subcore_axis_name="s")
    sc_params = pltpu.CompilerParams(use_tc_tiling_on_sc=False)
    src_feat = pl.kernel(
        _p2_body,
        out_type=jax.ShapeDtypeStruct((_E, _DI), f32),
        mesh=mesh,
        compiler_params=sc_params,
        scratch_types=[pltpu.VMEM_SHARED((_N, _DI), f32),
                       pltpu.VMEM((8, 128), jnp.int32),
                       pltpu.VMEM((512, _DI), f32),
                       pltpu.SemaphoreType.DMA],
    )(nf, edge_src)

    grid_e = _E // _BE
    espec = lambda w: pl.BlockSpec((_BE, w), lambda i: (i, 0))
    ef = pl.pallas_call(
        _p3_body,
        grid=(grid_e,),
        in_specs=[espec(_NB), espec(_DE), espec(_DI),
                  full(_NB, _DR), full(1, _DR), full(_DR, _DR), full(1, _DR),
                  full(_DR, _DM), full(1, _DM), full(_DI, _DM), full(_DE, _DM)],
        out_specs=[pl.BlockSpec((2, _BE, _DO), lambda i: (0, i, 0))],
        out_shape=[jax.ShapeDtypeStruct((2, _E, _DO), f32)],
    )(edge_length_embedded, edge_attr, src_feat,
      W1, b1r, W2, b2r, w3, b3s, rm, tm)[0]

    zrows = jnp.zeros((_ZR, _DO), f32)
    agg2 = pl.kernel(
        _p4_body,
        out_type=jax.ShapeDtypeStruct((2, 2, _NH, _DO), f32),
        mesh=plsc.VectorSubcoreMesh(core_axis_name="c", subcore_axis_name="s"),
        compiler_params=sc_params,
        scratch_types=[pltpu.VMEM_SHARED((_SPR, _DO), f32),
                       pltpu.VMEM((256, _DO), f32),
                       pltpu.VMEM((8, 128), jnp.int32)],
    )(ef, edge_dst, zrows)

    agg_a = agg2[0].reshape(_N, _DO)
    agg_b = agg2[1].reshape(_N, _DO)

    out = pl.pallas_call(
        _p5_body,
        grid=(grid_n,),
        in_specs=[nspec(_DO), nspec(_DO), nspec(_DA), nspec(_DO), nspec(1),
                  full(_DO, _DA * _DO), full(_DO, _DA * _DO)],
        out_specs=[nspec(_DO)],
        out_shape=[jax.ShapeDtypeStruct((_N, _DO), f32)],
    )(agg_a, agg_b, node_attr, nm, node_deg, wa, wb)[0]
    return out


# speedup vs baseline: 1.7901x; 1.3382x over previous
import math

import numpy as np
import jax
import jax.numpy as jnp
from jax import lax
from jax.experimental import pallas as pl
from jax.experimental.pallas import tpu as pltpu
from jax.experimental.pallas import tpu_sc as plsc

_N = 50000
_E = 800000
_DI = 32
_DA = 32
_DE = 4
_NB = 10
_DR = 64
_DO = 64
_DM = 128

_BN = 1000
_BE = 6400
_C = 1280
_NCHUNK = _E // _C
_ROWS = _E // 128
_NH = 25000
_TR = 512
_SPR = 25600
_ZR = 1600


def _p1_body(x_ref, a_ref, dg_ref, wl_ref, wm_ref, ael_ref, aem_ref,
             sl_ref, sm_ref, nf_ref, nm_ref):
    x = x_ref[...]
    a = a_ref[...]
    r = lax.rsqrt(dg_ref[...])
    gl = jnp.dot(x, wl_ref[...], preferred_element_type=jnp.float32)
    gm = jnp.dot(x, wm_ref[...], preferred_element_type=jnp.float32)
    al = jnp.dot(a, ael_ref[...], preferred_element_type=jnp.float32)
    am = jnp.dot(a, aem_ref[...], preferred_element_type=jnp.float32)
    nf_ref[...] = jnp.dot(gl * al, sl_ref[...],
                          preferred_element_type=jnp.float32) * r
    nm_ref[...] = jnp.dot(gm * am, sm_ref[...],
                          preferred_element_type=jnp.float32)


def _p2_body(nf_hbm, src_hbm, out_hbm, nf_sh, idx_ref, rows_ref, sem):
    c = lax.axis_index("c")
    s = lax.axis_index("s")
    wid = s * 2 + c
    @pl.when(s < 15)
    def _():
        pltpu.sync_copy(nf_hbm.at[pl.ds(s * 3128, 3128)],
                        nf_sh.at[pl.ds(s * 3128, 3128)])

    @pl.when(s == 15)
    def _():
        pltpu.sync_copy(nf_hbm.at[pl.ds(15 * 3128, _N - 15 * 3128)],
                        nf_sh.at[pl.ds(15 * 3128, _N - 15 * 3128)])

    plsc.subcore_barrier()
    nchunks = (_NCHUNK - wid + 31) // 32

    def chunk(k, carry):
        cid = wid + k * 32
        r0 = cid * 10
        e0 = cid * _C
        pltpu.sync_copy(src_hbm.at[pl.ds(r0, 10)], idx_ref)
        d = pltpu.async_copy(nf_sh.at[idx_ref.at[0]], rows_ref.at[0], sem)
        for q in range(10):
            dn = None
            if q < 9:
                dn = pltpu.async_copy(nf_sh.at[idx_ref.at[q + 1]],
                                      rows_ref.at[(q + 1) % 2], sem)
            d.wait()
            pltpu.sync_copy(rows_ref.at[q % 2],
                            out_hbm.at[pl.ds(e0 + q * 128, 128)])
            d = dn
        return carry

    lax.fori_loop(0, nchunks, chunk, 0)


def _p3_body(el_ref, ea_ref, sf_ref, w1_ref, b1_ref, w2_ref, b2_ref,
             w3_ref, b3_ref, rm_ref, tm_ref, ef_ref):
    h = jnp.dot(el_ref[...], w1_ref[...], preferred_element_type=jnp.float32) + b1_ref[...]
    h = h * jax.nn.sigmoid(h)
    h = jnp.dot(h, w2_ref[...], preferred_element_type=jnp.float32) + b2_ref[...]
    h = h * jax.nn.sigmoid(h)
    ew = jnp.dot(h, w3_ref[...], preferred_element_type=jnp.float32) + b3_ref[...]
    se = jnp.dot(sf_ref[...], rm_ref[...], preferred_element_type=jnp.float32)
    at = jnp.dot(ea_ref[...], tm_ref[...], preferred_element_type=jnp.float32)
    ef = se * at * ew
    ef_ref[0] = ef[:, :_DO]
    ef_ref[1] = ef[:, _DO:]


def _p3b_body(d_ref, i0_ref, i1_ref):
    d = d_ref[...]
    tr = _NH + (
        (lax.broadcasted_iota(jnp.int32, d.shape, 0) * 128
         + lax.broadcasted_iota(jnp.int32, d.shape, 1)) & (_TR - 1))
    i0_ref[...] = jnp.where(d < _NH, d, tr)
    i1_ref[...] = jnp.where(d >= _NH, d - _NH, tr)


def _p4_body(ef_hbm, i0_hbm, i1_hbm, z_hbm, out_hbm, acc_sh, ef_ref, idx_ref, sem):
    c = lax.axis_index("c")
    s = lax.axis_index("s")
    nchunks = (_NCHUNK - s + 15) // 16
    for p in range(2):
        idx_hbm = i0_hbm if p == 0 else i1_hbm
        pltpu.sync_copy(z_hbm, acc_sh.at[pl.ds(s * _ZR, _ZR)])
        plsc.subcore_barrier()

        def chunk(k, carry):
            cid = s + k * 16
            r0 = cid * 10
            e0 = cid * _C
            pltpu.sync_copy(idx_hbm.at[pl.ds(r0, 10)], idx_ref)
            d = pltpu.async_copy(ef_hbm.at[c, pl.ds(e0, 128)],
                                 ef_ref.at[0], sem)
            for q in range(10):
                dn = None
                if q < 9:
                    dn = pltpu.async_copy(
                        ef_hbm.at[c, pl.ds(e0 + (q + 1) * 128, 128)],
                        ef_ref.at[(q + 1) % 2], sem)
                d.wait()
                pltpu.sync_copy(ef_ref.at[q % 2],
                                acc_sh.at[idx_ref.at[q]], add=True)
                d = dn
            return carry

        lax.fori_loop(0, nchunks, chunk, 0)
        plsc.subcore_barrier()

        @pl.when(s < 15)
        def _():
            pltpu.sync_copy(acc_sh.at[pl.ds(s * _ZR, _ZR)],
                            out_hbm.at[c, p, pl.ds(s * _ZR, _ZR)])

        @pl.when(s == 15)
        def _():
            pltpu.sync_copy(acc_sh.at[pl.ds(15 * _ZR, _NH - 15 * _ZR)],
                            out_hbm.at[c, p, pl.ds(15 * _ZR, _NH - 15 * _ZR)])


def _p5_body(aa_ref, ab_ref, a_ref, nm_ref, dg_ref, wa_ref, wb_ref,
             aem_ref, sm_ref, o_ref):
    r = lax.rsqrt(dg_ref[...])
    g = (jnp.dot(aa_ref[...] * r, wa_ref[...], preferred_element_type=jnp.float32)
         + jnp.dot(ab_ref[...] * r, wb_ref[...], preferred_element_type=jnp.float32))
    am = jnp.dot(a_ref[...], aem_ref[...], preferred_element_type=jnp.float32)
    o_ref[...] = nm_ref[...] + jnp.dot(
        g * am, sm_ref[...], preferred_element_type=jnp.float32)


def kernel(node_input, node_attr, node_deg, edge_src, edge_dst, edge_attr,
           edge_length_embedded, W_lin_in, W_mask, W1, b1, W2, b2, W3, b3, W_out):
    f32 = jnp.float32
    c_s, c_x = math.sin(math.pi / 8), math.cos(math.pi / 8)

    wl = W_lin_in.reshape(_DI, _DA * _DI) * (1.0 / 32.0)
    wm = W_mask.reshape(_DI, _DA * _DO) * (c_s / 32.0)
    wo = W_out.reshape(_DM, _DA * _DO) * (c_x / 64.0)
    wa, wb = wo[:_DO], wo[_DO:]
    w3 = W3 * 0.5
    b3s = (b3 * 0.5).reshape(1, _DM)
    b1r = b1.reshape(1, _DR)
    b2r = b2.reshape(1, _DR)

    rm_np = np.zeros((_DI, _DM), np.float32)
    tm_np = np.zeros((_DE, _DM), np.float32)
    for i in range(_DI):
        rm_np[i, 4 * i:4 * i + 4] = 1.0
    for cc in range(_DE):
        tm_np[cc, cc::4] = 1.0
    ael_np = np.zeros((_DA, _DA * _DI), np.float32)
    aem_np = np.zeros((_DA, _DA * _DO), np.float32)
    for j in range(_DA):
        ael_np[j, j * _DI:(j + 1) * _DI] = 1.0
        aem_np[j, j * _DO:(j + 1) * _DO] = 1.0
    sl_np = np.tile(np.eye(_DI, dtype=np.float32), (_DA, 1))
    sm_np = np.tile(np.eye(_DO, dtype=np.float32), (_DA, 1))
    rm, tm = jnp.asarray(rm_np), jnp.asarray(tm_np)
    ael, aem = jnp.asarray(ael_np), jnp.asarray(aem_np)
    sl, sm = jnp.asarray(sl_np), jnp.asarray(sm_np)

    grid_n = _N // _BN
    nspec = lambda w: pl.BlockSpec((_BN, w), lambda i: (i, 0))
    full = lambda *shp: pl.BlockSpec(shp, lambda i: tuple(0 for _ in shp))

    nf, nm = pl.pallas_call(
        _p1_body,
        grid=(grid_n,),
        in_specs=[nspec(_DI), nspec(_DA), nspec(1),
                  full(_DI, _DA * _DI), full(_DI, _DA * _DO),
                  full(_DA, _DA * _DI), full(_DA, _DA * _DO),
                  full(_DA * _DI, _DI), full(_DA * _DO, _DO)],
        out_specs=[nspec(_DI), nspec(_DO)],
        out_shape=[jax.ShapeDtypeStruct((_N, _DI), f32),
                   jax.ShapeDtypeStruct((_N, _DO), f32)],
    )(node_input, node_attr, node_deg, wl, wm, ael, aem, sl, sm)

    mesh = plsc.VectorSubcoreMesh(core_axis_name="c", subcore_axis_name="s")
    sc_params = pltpu.CompilerParams(use_tc_tiling_on_sc=False)
    src2d = edge_src.reshape(_ROWS, 128)
    src_feat = pl.kernel(
        _p2_body,
        out_type=jax.ShapeDtypeStruct((_E, _DI), f32),
        mesh=mesh,
        compiler_params=sc_params,
        scratch_types=[pltpu.VMEM_SHARED((_N, _DI), f32),
                       pltpu.VMEM((10, 128), jnp.int32),
                       pltpu.VMEM((2, 128, _DI), f32),
                       pltpu.SemaphoreType.DMA],
    )(nf, src2d)

    grid_e = _E // _BE
    espec = lambda w: pl.BlockSpec((_BE, w), lambda i: (i, 0))
    ef = pl.pallas_call(
        _p3_body,
        grid=(grid_e,),
        in_specs=[espec(_NB), espec(_DE), espec(_DI),
                  full(_NB, _DR), full(1, _DR), full(_DR, _DR), full(1, _DR),
                  full(_DR, _DM), full(1, _DM), full(_DI, _DM), full(_DE, _DM)],
        out_specs=[pl.BlockSpec((2, _BE, _DO), lambda i: (0, i, 0))],
        out_shape=[jax.ShapeDtypeStruct((2, _E, _DO), f32)],
    )(edge_length_embedded, edge_attr, src_feat,
      W1, b1r, W2, b2r, w3, b3s, rm, tm)[0]

    dst2d = edge_dst.reshape(_ROWS, 128)
    i0, i1 = pl.pallas_call(
        _p3b_body,
        grid=(1,),
        in_specs=[pl.BlockSpec((_ROWS, 128), lambda i: (0, 0))],
        out_specs=[pl.BlockSpec((_ROWS, 128), lambda i: (0, 0)),
                   pl.BlockSpec((_ROWS, 128), lambda i: (0, 0))],
        out_shape=[jax.ShapeDtypeStruct((_ROWS, 128), jnp.int32),
                   jax.ShapeDtypeStruct((_ROWS, 128), jnp.int32)],
    )(dst2d)

    zrows = jnp.zeros((_ZR, _DO), f32)
    agg2 = pl.kernel(
        _p4_body,
        out_type=jax.ShapeDtypeStruct((2, 2, _NH, _DO), f32),
        mesh=plsc.VectorSubcoreMesh(core_axis_name="c", subcore_axis_name="s"),
        compiler_params=sc_params,
        scratch_types=[pltpu.VMEM_SHARED((_SPR, _DO), f32),
                       pltpu.VMEM((2, 128, _DO), f32),
                       pltpu.VMEM((10, 128), jnp.int32),
                       pltpu.SemaphoreType.DMA],
    )(ef, i0, i1, zrows)

    agg_a = agg2[0].reshape(_N, _DO)
    agg_b = agg2[1].reshape(_N, _DO)

    out = pl.pallas_call(
        _p5_body,
        grid=(grid_n,),
        in_specs=[nspec(_DO), nspec(_DO), nspec(_DA), nspec(_DO), nspec(1),
                  full(_DO, _DA * _DO), full(_DO, _DA * _DO),
                  full(_DA, _DA * _DO), full(_DA * _DO, _DO)],
        out_specs=[nspec(_DO)],
        out_shape=[jax.ShapeDtypeStruct((_N, _DO), f32)],
    )(agg_a, agg_b, node_attr, nm, node_deg, wa, wb, aem, sm)[0]
    return out
